# Initial kernel scaffold; baseline (speedup 1.0000x reference)
import jax
import jax.numpy as jnp
from jax.experimental import pallas as pl


def kernel(mu, scale, rotation, features, voxel_coords, point_cloud_range):
    raise NotImplementedError("write your pallas kernel here")


# speedup vs baseline: 1.2797x; 1.2797x over previous
import jax
import jax.numpy as jnp
from jax.experimental import pallas as pl
from jax.experimental.pallas import tpu as pltpu

VOXEL_SIZE = (0.05, 0.05, 0.05)
SCALE_MULTIPLIER = 3.0
MAX_NEIGHBORS = 64
N_GAUSS = 8192
M_VOXELS = 16384

ROWS = 256
GB = 1024
XROWS = 8


def _score_kernel(vc_ref, mu_ref, sc_ref, pcr_ref, scores_ref, cent_ref):
    x = vc_ref[:, 3:4].astype(jnp.float32)
    y = vc_ref[:, 2:3].astype(jnp.float32)
    z = vc_ref[:, 1:2].astype(jnp.float32)
    cx = pcr_ref[0, 0] + (x + 0.5) * VOXEL_SIZE[0]
    cy = pcr_ref[0, 1] + (y + 0.5) * VOXEL_SIZE[1]
    cz = pcr_ref[0, 2] + (z + 0.5) * VOXEL_SIZE[2]
    cent_ref[...] = jnp.concatenate([cx, cy, cz], axis=1)

    mux = mu_ref[0:1, :]
    muy = mu_ref[1:2, :]
    muz = mu_ref[2:3, :]
    sx = sc_ref[0:1, :]
    sy = sc_ref[1:2, :]
    sz = sc_ref[2:3, :]
    s2x = sx * sx
    s2y = sy * sy
    s2z = sz * sz
    ivx = 1.0 / (s2x + 1e-8)
    ivy = 1.0 / (s2y + 1e-8)
    ivz = 1.0 / (s2z + 1e-8)
    r = SCALE_MULTIPLIER * jnp.sqrt(s2x + s2y + s2z)
    r2 = r * r

    dx = cx - mux
    dy = cy - muy
    dz = cz - muz
    d2 = dx * dx * ivx + dy * dy * ivy + dz * dz * ivz
    scores_ref[...] = jnp.where(d2 <= r2, d2, jnp.inf)


def _extract_kernel(scores_ref, idx_ref, w_ref):
    s0 = scores_ref[...]
    iota_l = jax.lax.broadcasted_iota(jnp.int32, (XROWS, N_GAUSS), 1)
    lane64 = jax.lax.broadcasted_iota(jnp.int32, (XROWS, MAX_NEIGHBORS), 1)
    BIG = jnp.int32(2**30)

    def body(k, carry):
        outi, outw, s = carry
        m = jnp.min(s, axis=1, keepdims=True)
        idx = jnp.min(jnp.where(s == m, iota_l, BIG), axis=1, keepdims=True)
        s = jnp.where(iota_l == idx, jnp.inf, s)
        outi = jnp.where(lane64 == k, idx, outi)
        outw = jnp.where(lane64 == k, m, outw)
        return outi, outw, s

    outi = jnp.zeros((XROWS, MAX_NEIGHBORS), jnp.int32)
    outw = jnp.full((XROWS, MAX_NEIGHBORS), jnp.inf, jnp.float32)
    outi, outw, _ = jax.lax.fori_loop(0, MAX_NEIGHBORS, body, (outi, outw, s0))
    valid = outw < jnp.inf
    idx_ref[...] = jnp.where(valid, outi, -1)
    w_ref[...] = jnp.where(valid, outw, 0.0)


def kernel(mu, scale, rotation, features, voxel_coords, point_cloud_range):
    vc = voxel_coords.astype(jnp.int32)
    mu_t = mu.T
    sc_t = scale.T
    pcr = point_cloud_range.reshape(1, 6)

    scores, centers = pl.pallas_call(
        _score_kernel,
        grid=(M_VOXELS // ROWS, N_GAUSS // GB),
        in_specs=[
            pl.BlockSpec((ROWS, 4), lambda i, j: (i, 0)),
            pl.BlockSpec((3, GB), lambda i, j: (0, j)),
            pl.BlockSpec((3, GB), lambda i, j: (0, j)),
            pl.BlockSpec((1, 6), lambda i, j: (0, 0)),
        ],
        out_specs=[
            pl.BlockSpec((ROWS, GB), lambda i, j: (i, j)),
            pl.BlockSpec((ROWS, 3), lambda i, j: (i, 0)),
        ],
        out_shape=[
            jax.ShapeDtypeStruct((M_VOXELS, N_GAUSS), jnp.float32),
            jax.ShapeDtypeStruct((M_VOXELS, 3), jnp.float32),
        ],
    )(vc, mu_t, sc_t, pcr)

    idx, w = pl.pallas_call(
        _extract_kernel,
        grid=(M_VOXELS // XROWS,),
        in_specs=[pl.BlockSpec((XROWS, N_GAUSS), lambda i: (i, 0))],
        out_specs=[
            pl.BlockSpec((XROWS, MAX_NEIGHBORS), lambda i: (i, 0)),
            pl.BlockSpec((XROWS, MAX_NEIGHBORS), lambda i: (i, 0)),
        ],
        out_shape=[
            jax.ShapeDtypeStruct((M_VOXELS, MAX_NEIGHBORS), jnp.int32),
            jax.ShapeDtypeStruct((M_VOXELS, MAX_NEIGHBORS), jnp.float32),
        ],
    )(scores)

    masks = idx >= 0
    return centers, idx, w, masks


# speedup vs baseline: 25.0883x; 19.6046x over previous
import functools

import jax
import jax.numpy as jnp
from jax import lax
from jax.experimental import pallas as pl
from jax.experimental.pallas import tpu as pltpu
from jax.experimental.pallas import tpu_sc as plsc

VOXEL_SIZE = (0.05, 0.05, 0.05)
SCALE_MULTIPLIER = 3.0
MAX_NEIGHBORS = 64
N_GAUSS = 8192
M_VOXELS = 16384

ROWS = 256
GB = 1024
NBLK = 64
INF = float("inf")

NWORK = 32
RW = M_VOXELS // NWORK
P = 256
NPASS = RW // P
WAVE = 32
NRING = 4
CANDCAP = N_GAUSS + 64


def _score_kernel(vc_ref, mu_ref, sc_ref, pcr_ref, scores_ref, bm_ref, cent_ref):
    x = vc_ref[:, 3:4].astype(jnp.float32)
    y = vc_ref[:, 2:3].astype(jnp.float32)
    z = vc_ref[:, 1:2].astype(jnp.float32)
    cx = pcr_ref[0, 0] + (x + 0.5) * VOXEL_SIZE[0]
    cy = pcr_ref[0, 1] + (y + 0.5) * VOXEL_SIZE[1]
    cz = pcr_ref[0, 2] + (z + 0.5) * VOXEL_SIZE[2]
    cent_ref[...] = jnp.concatenate([cx, cy, cz], axis=1)

    mux = mu_ref[0:1, :]
    muy = mu_ref[1:2, :]
    muz = mu_ref[2:3, :]
    sx = sc_ref[0:1, :]
    sy = sc_ref[1:2, :]
    sz = sc_ref[2:3, :]
    s2x = sx * sx
    s2y = sy * sy
    s2z = sz * sz
    ivx = 1.0 / (s2x + 1e-8)
    ivy = 1.0 / (s2y + 1e-8)
    ivz = 1.0 / (s2z + 1e-8)
    r = SCALE_MULTIPLIER * jnp.sqrt(s2x + s2y + s2z)
    r2 = r * r

    dx = cx - mux
    dy = cy - muy
    dz = cz - muz
    d2 = dx * dx * ivx + dy * dy * ivy + dz * dz * ivz
    md2 = jnp.where(d2 <= r2, d2, INF)
    scores_ref[...] = md2
    mins = [jnp.min(md2[:, k * 128:(k + 1) * 128], axis=1, keepdims=True)
            for k in range(GB // 128)]
    bm_ref[...] = jnp.concatenate(mins, axis=1)[None]


def _splat_i32(x):
    return jnp.full((16,), x, jnp.int32)


def _splat_f32(x):
    return jnp.full((16,), x, jnp.float32)


def _mask_i32(mask):
    return jnp.where(mask, _splat_i32(1), _splat_i32(0))


def _vsort(k, v):
    return plsc.sort_key_val(k, v)


def _rev2(k, v):
    return lax.rev(k, (0,)), lax.rev(v, (0,))


def _cmpsel(ak, av, bk, bv):
    m = ak <= bk
    lk = jnp.where(m, ak, bk)
    lv = jnp.where(m, av, bv)
    hk = jnp.where(m, bk, ak)
    hv = jnp.where(m, bv, av)
    return lk, lv, hk, hv


def _merge16(ak, av, bk, bv):
    rbk, rbv = _rev2(bk, bv)
    lk, lv, hk, hv = _cmpsel(ak, av, rbk, rbv)
    lk, lv = _vsort(lk, lv)
    hk, hv = _vsort(hk, hv)
    return lk, lv, hk, hv


def _bitonic2(x0k, x0v, x1k, x1v):
    lk, lv, hk, hv = _cmpsel(x0k, x0v, x1k, x1v)
    lk, lv = _vsort(lk, lv)
    hk, hv = _vsort(hk, hv)
    return lk, lv, hk, hv


def _sort64(k, v):
    s = [_vsort(k[i], v[i]) for i in range(4)]
    a0k, a0v, a1k, a1v = _merge16(*s[0], *s[1])
    b0k, b0v, b1k, b1v = _merge16(*s[2], *s[3])
    rb0k, rb0v = _rev2(b1k, b1v)
    rb1k, rb1v = _rev2(b0k, b0v)
    l0k, l0v, h0k, h0v = _cmpsel(a0k, a0v, rb0k, rb0v)
    l1k, l1v, h1k, h1v = _cmpsel(a1k, a1v, rb1k, rb1v)
    s0k, s0v, s1k, s1v = _bitonic2(l0k, l0v, l1k, l1v)
    s2k, s2v, s3k, s3v = _bitonic2(h0k, h0v, h1k, h1v)
    return (s0k, s1k, s2k, s3k), (s0v, s1v, s2v, s3v)


def _insert16(sk, sv, tk, tv):
    rtk, rtv = _rev2(tk, tv)
    mk, mv, _, _ = _cmpsel(sk[3], sv[3], rtk, rtv)
    t2k, t2v = _vsort(mk, mv)
    r2k, r2v = _rev2(t2k, t2v)
    lk, lv, hk, hv = _cmpsel(sk[2], sv[2], r2k, r2v)
    n3k, n3v = _vsort(hk, hv)
    t3k, t3v = _vsort(lk, lv)
    r3k, r3v = _rev2(t3k, t3v)
    lk, lv, hk, hv = _cmpsel(sk[1], sv[1], r3k, r3v)
    n2k, n2v = _vsort(hk, hv)
    t4k, t4v = _vsort(lk, lv)
    r4k, r4v = _rev2(t4k, t4v)
    lk, lv, hk, hv = _cmpsel(sk[0], sv[0], r4k, r4v)
    n1k, n1v = _vsort(hk, hv)
    n0k, n0v = _vsort(lk, lv)
    return (n0k, n1k, n2k, n3k), (n0v, n1v, n2v, n3v)


def _popcount(mask):
    return plsc.all_reduce_population_count(mask)[0]


def _sc_select_body(scores_hbm, bm_hbm, oi_hbm, ow_hbm,
                    bm_v, list_v, rb0, rb1, rb2, rb3,
                    ck_v, ci_v, kk_v, vv_v, si_v, sw_v,
                    sem0, sem1, sem2, sem3):
    rings = (rb0, rb1, rb2, rb3)
    sems = (sem0, sem1, sem2, sem3)
    wid = lax.axis_index("s") * 2 + lax.axis_index("c")

    def finalize(cur_row, cnt, pass_base):
        r_local = cur_row - pass_base
        for q in range(4):
            ck_v[pl.ds(cnt + q * 16, 16)] = jnp.full((16,), INF, jnp.float32)
        k4 = [ck_v[pl.ds(i * 16, 16)] for i in range(4)]
        v4 = [ci_v[pl.ds(i * 16, 16)] for i in range(4)]
        sk, sv = _sort64(k4, v4)
        nch = jnp.maximum(0, (cnt - 49) // 16)
        for j in range(4):
            kk_v[pl.ds(j * 16, 16)] = sk[j]
            vv_v[pl.ds(j * 16, 16)] = sv[j]

        def chunk_body(m, c):
            off = 64 + m * 16
            tk, tv = _vsort(ck_v[pl.ds(off, 16)], ci_v[pl.ds(off, 16)])
            csk = [kk_v[pl.ds(j * 16, 16)] for j in range(4)]
            csv = [vv_v[pl.ds(j * 16, 16)] for j in range(4)]
            nk, nv = _insert16(csk, csv, tk, tv)
            for j in range(4):
                kk_v[pl.ds(j * 16, 16)] = nk[j]
                vv_v[pl.ds(j * 16, 16)] = nv[j]
            return c
        lax.fori_loop(0, nch, chunk_body, 0)

        for j in range(4):
            fk = kk_v[pl.ds(j * 16, 16)]
            fv = vv_v[pl.ds(j * 16, 16)]
            valid = fk < INF
            si_v[pl.ds(r_local * 64 + j * 16, 16)] = jnp.where(
                valid, fv, jnp.full((16,), -1, jnp.int32))
            sw_v[pl.ds(r_local * 64 + j * 16, 16)] = jnp.where(
                valid, fk, jnp.full((16,), 0.0, jnp.float32))

    def pass_body(pi, _):
        pass_base = wid * RW + pi * P

        def init_body(t, c):
            si_v[pl.ds(t * 16, 16)] = jnp.full((16,), -1, jnp.int32)
            sw_v[pl.ds(t * 16, 16)] = jnp.zeros((16,), jnp.float32)
            return c
        lax.fori_loop(0, P * 64 // 16, init_body, 0)

        pltpu.sync_copy(bm_hbm.at[pl.ds(pass_base * 64, P * 64)], bm_v)

        def brow(r, off):
            for j in range(4):
                bmv = bm_v[pl.ds(r * 64 + j * 16, 16)]
                mask = bmv < INF
                pc = _popcount(mask)
                ids = ((pass_base + r) * 64 + j * 16
                       + lax.broadcasted_iota(jnp.int32, (16,), 0))
                plsc.store_compressed(list_v.at[pl.ds(off, 16)], ids,
                                      mask=mask)
                off = off + pc
            return off

        num_list = lax.fori_loop(0, P, brow, jnp.int32(0))
        list_v[pl.ds(num_list, 16)] = jnp.zeros((16,), jnp.int32)
        list_v[pl.ds(num_list + 16, 16)] = jnp.zeros((16,), jnp.int32)
        nw = (num_list + WAVE - 1) // WAVE

        for s in range(NRING):
            @pl.when(s < nw)
            def _():
                pltpu.async_copy(
                    scores_hbm.at[list_v.at[pl.ds(s * WAVE, WAVE)]],
                    rings[s], sems[s])

        def group_body(g, carry):
            for s in range(NRING):
                w = g * NRING + s

                def do_wave(c, s=s, w=w):
                    pltpu.make_async_copy(
                        scores_hbm.at[list_v.at[pl.ds(0, WAVE)]],
                        rings[s], sems[s]).wait()
                    nblk = jnp.minimum(WAVE, num_list - w * WAVE)

                    def blk_body(b, cc):
                        cur_row, cnt = cc
                        bid = list_v[pl.ds(w * WAVE + b, 16)][0]
                        row = bid // 64
                        blk = bid % 64

                        def rowchange(cr, cn):
                            @pl.when(cn > 0)
                            def _():
                                finalize(cr, cn, pass_base)
                            return row, jnp.int32(0)

                        cur_row, cnt = lax.cond(row != cur_row, rowchange,
                                                lambda cr, cn: (cr, cn),
                                                cur_row, cnt)
                        rowref = rings[s].at[b]
                        for v8 in range(8):
                            xv = rowref[pl.ds(v8 * 16, 16)]
                            mask = xv < INF
                            pc = _popcount(mask)
                            gidx = (blk * 128 + v8 * 16
                                    + lax.broadcasted_iota(jnp.int32, (16,), 0))
                            plsc.store_compressed(ck_v.at[pl.ds(cnt, 16)], xv,
                                                  mask=mask)
                            plsc.store_compressed(ci_v.at[pl.ds(cnt, 16)],
                                                  gidx, mask=mask)
                            cnt = cnt + pc
                        return cur_row, cnt

                    c = lax.fori_loop(0, nblk, blk_body, c)

                    @pl.when(w + NRING < nw)
                    def _():
                        pltpu.async_copy(
                            scores_hbm.at[
                                list_v.at[pl.ds((w + NRING) * WAVE, WAVE)]],
                            rings[s], sems[s])
                    return c

                carry = lax.cond(w < nw, do_wave, lambda c: c, carry)
            return carry

        ngroups = (nw + NRING - 1) // NRING
        cur_row, cnt = lax.fori_loop(0, ngroups, group_body,
                                     (pass_base, jnp.int32(0)))

        @pl.when(cnt > 0)
        def _():
            finalize(cur_row, cnt, pass_base)

        pltpu.sync_copy(si_v, oi_hbm.at[pl.ds(pass_base * 64, P * 64)])
        pltpu.sync_copy(sw_v, ow_hbm.at[pl.ds(pass_base * 64, P * 64)])
        return 0

    lax.fori_loop(0, NPASS, pass_body, 0)


def _build_sc_select():
    return functools.partial(
        pl.kernel,
        out_type=[
            jax.ShapeDtypeStruct((M_VOXELS * MAX_NEIGHBORS,), jnp.int32),
            jax.ShapeDtypeStruct((M_VOXELS * MAX_NEIGHBORS,), jnp.float32),
        ],
        mesh=plsc.VectorSubcoreMesh(core_axis_name="c", subcore_axis_name="s",
                                    num_cores=2, num_subcores=16),
        compiler_params=pltpu.CompilerParams(needs_layout_passes=False),
        scratch_types=[
            pltpu.VMEM((P * NBLK,), jnp.float32),
            pltpu.VMEM((P * NBLK + 2 * 16,), jnp.int32),
            pltpu.VMEM((WAVE, 128), jnp.float32),
            pltpu.VMEM((WAVE, 128), jnp.float32),
            pltpu.VMEM((WAVE, 128), jnp.float32),
            pltpu.VMEM((WAVE, 128), jnp.float32),
            pltpu.VMEM((CANDCAP,), jnp.float32),
            pltpu.VMEM((CANDCAP,), jnp.int32),
            pltpu.VMEM((64,), jnp.float32),
            pltpu.VMEM((64,), jnp.int32),
            pltpu.VMEM((P * MAX_NEIGHBORS,), jnp.int32),
            pltpu.VMEM((P * MAX_NEIGHBORS,), jnp.float32),
            pltpu.SemaphoreType.DMA,
            pltpu.SemaphoreType.DMA,
            pltpu.SemaphoreType.DMA,
            pltpu.SemaphoreType.DMA,
        ],
    )(_sc_select_body)


def kernel(mu, scale, rotation, features, voxel_coords, point_cloud_range):
    vc = voxel_coords.astype(jnp.int32)
    mu_t = mu.T
    sc_t = scale.T
    pcr = point_cloud_range.reshape(1, 6)

    scores, bm, centers = pl.pallas_call(
        _score_kernel,
        grid=(M_VOXELS // ROWS, N_GAUSS // GB),
        in_specs=[
            pl.BlockSpec((ROWS, 4), lambda i, j: (i, 0)),
            pl.BlockSpec((3, GB), lambda i, j: (0, j)),
            pl.BlockSpec((3, GB), lambda i, j: (0, j)),
            pl.BlockSpec((1, 6), lambda i, j: (0, 0)),
        ],
        out_specs=[
            pl.BlockSpec((ROWS, GB), lambda i, j: (i, j)),
            pl.BlockSpec((1, ROWS, GB // 128), lambda i, j: (j, i, 0)),
            pl.BlockSpec((ROWS, 3), lambda i, j: (i, 0)),
        ],
        out_shape=[
            jax.ShapeDtypeStruct((M_VOXELS, N_GAUSS), jnp.float32),
            jax.ShapeDtypeStruct((N_GAUSS // GB, M_VOXELS, GB // 128),
                                 jnp.float32),
            jax.ShapeDtypeStruct((M_VOXELS, 3), jnp.float32),
        ],
    )(vc, mu_t, sc_t, pcr)

    bm_flat = jnp.transpose(bm, (1, 0, 2)).reshape(M_VOXELS * NBLK)
    oi, ow = _build_sc_select()(scores.reshape(M_VOXELS * NBLK, 128),
                                bm_flat)
    idx = oi.reshape(M_VOXELS, MAX_NEIGHBORS)
    w = ow.reshape(M_VOXELS, MAX_NEIGHBORS)
    masks = idx >= 0
    return centers, idx, w, masks


# speedup vs baseline: 31.1372x; 1.2411x over previous
import functools

import jax
import jax.numpy as jnp
from jax import lax
from jax.experimental import pallas as pl
from jax.experimental.pallas import tpu as pltpu
from jax.experimental.pallas import tpu_sc as plsc

VOXEL_SIZE = (0.05, 0.05, 0.05)
SCALE_MULTIPLIER = 3.0
MAX_NEIGHBORS = 64
N_GAUSS = 8192
M_VOXELS = 16384

ROWS = 256
GB = 1024
NBLK = 64
INF = float("inf")

NWORK = 32
RW = M_VOXELS // NWORK
P = 256
NPASS = RW // P
WAVE = 32
NRING = 4
CANDCAP = N_GAUSS + 64


def _score_kernel(vc_ref, mu_ref, sc_ref, pcr_ref, scores_ref, bm_ref, cent_ref):
    x = vc_ref[:, 3:4].astype(jnp.float32)
    y = vc_ref[:, 2:3].astype(jnp.float32)
    z = vc_ref[:, 1:2].astype(jnp.float32)
    cx = pcr_ref[0, 0] + (x + 0.5) * VOXEL_SIZE[0]
    cy = pcr_ref[0, 1] + (y + 0.5) * VOXEL_SIZE[1]
    cz = pcr_ref[0, 2] + (z + 0.5) * VOXEL_SIZE[2]
    cent_ref[...] = jnp.concatenate([cx, cy, cz], axis=1)

    mux = mu_ref[0:1, :]
    muy = mu_ref[1:2, :]
    muz = mu_ref[2:3, :]
    sx = sc_ref[0:1, :]
    sy = sc_ref[1:2, :]
    sz = sc_ref[2:3, :]
    s2x = sx * sx
    s2y = sy * sy
    s2z = sz * sz
    ivx = 1.0 / (s2x + 1e-8)
    ivy = 1.0 / (s2y + 1e-8)
    ivz = 1.0 / (s2z + 1e-8)
    r = SCALE_MULTIPLIER * jnp.sqrt(s2x + s2y + s2z)
    r2 = r * r

    dx = cx - mux
    dy = cy - muy
    dz = cz - muz
    d2 = dx * dx * ivx + dy * dy * ivy + dz * dz * ivz
    md2 = jnp.where(d2 <= r2, d2, INF)
    scores_ref[...] = md2
    mins = [jnp.min(md2[:, k * 128:(k + 1) * 128], axis=1, keepdims=True)
            for k in range(GB // 128)]
    bm_ref[...] = jnp.concatenate(mins, axis=1)[None]


def _splat_i32(x):
    return jnp.full((16,), x, jnp.int32)


def _splat_f32(x):
    return jnp.full((16,), x, jnp.float32)


def _mask_i32(mask):
    return jnp.where(mask, _splat_i32(1), _splat_i32(0))


def _vsort(k, v):
    return plsc.sort_key_val(k, v)


def _rev2(k, v):
    return lax.rev(k, (0,)), lax.rev(v, (0,))


def _cmpsel(ak, av, bk, bv):
    m = ak <= bk
    lk = jnp.where(m, ak, bk)
    lv = jnp.where(m, av, bv)
    hk = jnp.where(m, bk, ak)
    hv = jnp.where(m, bv, av)
    return lk, lv, hk, hv


def _merge16(ak, av, bk, bv):
    rbk, rbv = _rev2(bk, bv)
    lk, lv, hk, hv = _cmpsel(ak, av, rbk, rbv)
    lk, lv = _vsort(lk, lv)
    hk, hv = _vsort(hk, hv)
    return lk, lv, hk, hv


def _bitonic2(x0k, x0v, x1k, x1v):
    lk, lv, hk, hv = _cmpsel(x0k, x0v, x1k, x1v)
    lk, lv = _vsort(lk, lv)
    hk, hv = _vsort(hk, hv)
    return lk, lv, hk, hv


def _sort64(k, v):
    s = [_vsort(k[i], v[i]) for i in range(4)]
    a0k, a0v, a1k, a1v = _merge16(*s[0], *s[1])
    b0k, b0v, b1k, b1v = _merge16(*s[2], *s[3])
    rb0k, rb0v = _rev2(b1k, b1v)
    rb1k, rb1v = _rev2(b0k, b0v)
    l0k, l0v, h0k, h0v = _cmpsel(a0k, a0v, rb0k, rb0v)
    l1k, l1v, h1k, h1v = _cmpsel(a1k, a1v, rb1k, rb1v)
    s0k, s0v, s1k, s1v = _bitonic2(l0k, l0v, l1k, l1v)
    s2k, s2v, s3k, s3v = _bitonic2(h0k, h0v, h1k, h1v)
    return (s0k, s1k, s2k, s3k), (s0v, s1v, s2v, s3v)


def _insert16(sk, sv, tk, tv):
    rtk, rtv = _rev2(tk, tv)
    mk, mv, _, _ = _cmpsel(sk[3], sv[3], rtk, rtv)
    t2k, t2v = _vsort(mk, mv)
    r2k, r2v = _rev2(t2k, t2v)
    lk, lv, hk, hv = _cmpsel(sk[2], sv[2], r2k, r2v)
    n3k, n3v = _vsort(hk, hv)
    t3k, t3v = _vsort(lk, lv)
    r3k, r3v = _rev2(t3k, t3v)
    lk, lv, hk, hv = _cmpsel(sk[1], sv[1], r3k, r3v)
    n2k, n2v = _vsort(hk, hv)
    t4k, t4v = _vsort(lk, lv)
    r4k, r4v = _rev2(t4k, t4v)
    lk, lv, hk, hv = _cmpsel(sk[0], sv[0], r4k, r4v)
    n1k, n1v = _vsort(hk, hv)
    n0k, n0v = _vsort(lk, lv)
    return (n0k, n1k, n2k, n3k), (n0v, n1v, n2v, n3v)


def _popcount(mask):
    return plsc.all_reduce_population_count(mask)[0]


def _sc_select_body(scores_hbm, bm_hbm, oi_hbm, ow_hbm,
                    bm_v, list_v, rb0, rb1, rb2, rb3,
                    ck_v, ci_v, kk_v, vv_v, si_v, sw_v,
                    sem0, sem1, sem2, sem3):
    rings = (rb0, rb1, rb2, rb3)
    sems = (sem0, sem1, sem2, sem3)
    wid = lax.axis_index("s") * 2 + lax.axis_index("c")

    def finalize(cur_row, cnt, pass_base):
        r_local = cur_row - pass_base
        for q in range(4):
            ck_v[pl.ds(cnt + q * 16, 16)] = jnp.full((16,), INF, jnp.float32)
        k4 = [ck_v[pl.ds(i * 16, 16)] for i in range(4)]
        v4 = [ci_v[pl.ds(i * 16, 16)] for i in range(4)]
        sk, sv = _sort64(k4, v4)
        nch = jnp.maximum(0, (cnt - 49) // 16)
        for j in range(4):
            kk_v[pl.ds(j * 16, 16)] = sk[j]
            vv_v[pl.ds(j * 16, 16)] = sv[j]

        def chunk_body(m, c):
            off = 64 + m * 16
            tk, tv = _vsort(ck_v[pl.ds(off, 16)], ci_v[pl.ds(off, 16)])
            csk = [kk_v[pl.ds(j * 16, 16)] for j in range(4)]
            csv = [vv_v[pl.ds(j * 16, 16)] for j in range(4)]
            nk, nv = _insert16(csk, csv, tk, tv)
            for j in range(4):
                kk_v[pl.ds(j * 16, 16)] = nk[j]
                vv_v[pl.ds(j * 16, 16)] = nv[j]
            return c
        lax.fori_loop(0, nch, chunk_body, 0)

        for j in range(4):
            fk = kk_v[pl.ds(j * 16, 16)]
            fv = vv_v[pl.ds(j * 16, 16)]
            valid = fk < INF
            si_v[pl.ds(r_local * 64 + j * 16, 16)] = jnp.where(
                valid, fv, jnp.full((16,), -1, jnp.int32))
            sw_v[pl.ds(r_local * 64 + j * 16, 16)] = jnp.where(
                valid, fk, jnp.full((16,), 0.0, jnp.float32))

    def pass_body(pi, _):
        pass_base = wid * RW + pi * P

        def init_body(t, c):
            si_v[pl.ds(t * 16, 16)] = jnp.full((16,), -1, jnp.int32)
            sw_v[pl.ds(t * 16, 16)] = jnp.zeros((16,), jnp.float32)
            return c
        lax.fori_loop(0, P * 64 // 16, init_body, 0)

        pltpu.sync_copy(bm_hbm.at[pl.ds(pass_base * 64, P * 64)], bm_v)

        def brow(r, off):
            for j in range(4):
                bmv = bm_v[pl.ds(r * 64 + j * 16, 16)]
                mask = bmv < INF
                pc = _popcount(mask)
                ids = ((pass_base + r) * 64 + j * 16
                       + lax.broadcasted_iota(jnp.int32, (16,), 0))
                plsc.store_compressed(list_v.at[pl.ds(off, 16)], ids,
                                      mask=mask)
                off = off + pc
            return off

        num_list = lax.fori_loop(0, P, brow, jnp.int32(0))
        list_v[pl.ds(num_list, 16)] = jnp.zeros((16,), jnp.int32)
        list_v[pl.ds(num_list + 16, 16)] = jnp.zeros((16,), jnp.int32)
        nw = (num_list + WAVE - 1) // WAVE

        for s in range(NRING):
            @pl.when(s < nw)
            def _():
                pltpu.async_copy(
                    scores_hbm.at[list_v.at[pl.ds(s * WAVE, WAVE)]],
                    rings[s], sems[s])

        def group_body(g, carry):
            for s in range(NRING):
                w = g * NRING + s

                def do_wave(c, s=s, w=w):
                    pltpu.make_async_copy(
                        scores_hbm.at[list_v.at[pl.ds(0, WAVE)]],
                        rings[s], sems[s]).wait()
                    nblk = jnp.minimum(WAVE, num_list - w * WAVE)

                    def blk_body(b, cc):
                        cur_row, cnt = cc
                        bid = list_v[pl.ds(w * WAVE + b, 16)][0]
                        row = bid // 64
                        blk = bid % 64

                        def rowchange(cr, cn):
                            @pl.when(cn > 0)
                            def _():
                                finalize(cr, cn, pass_base)
                            return row, jnp.int32(0)

                        cur_row, cnt = lax.cond(row != cur_row, rowchange,
                                                lambda cr, cn: (cr, cn),
                                                cur_row, cnt)
                        rowref = rings[s].at[b]
                        xvs = []
                        msks = []
                        offs = [cnt]
                        for v8 in range(8):
                            xv = rowref[pl.ds(v8 * 16, 16)]
                            mask = xv < INF
                            xvs.append(xv)
                            msks.append(mask)
                            offs.append(offs[-1] + _popcount(mask))
                        for v8 in range(8):
                            gidx = (blk * 128 + v8 * 16
                                    + lax.broadcasted_iota(jnp.int32, (16,), 0))
                            plsc.store_compressed(ck_v.at[pl.ds(offs[v8], 16)],
                                                  xvs[v8], mask=msks[v8])
                            plsc.store_compressed(ci_v.at[pl.ds(offs[v8], 16)],
                                                  gidx, mask=msks[v8])
                        return cur_row, offs[8]

                    c = lax.fori_loop(0, nblk, blk_body, c)

                    @pl.when(w + NRING < nw)
                    def _():
                        pltpu.async_copy(
                            scores_hbm.at[
                                list_v.at[pl.ds((w + NRING) * WAVE, WAVE)]],
                            rings[s], sems[s])
                    return c

                carry = lax.cond(w < nw, do_wave, lambda c: c, carry)
            return carry

        ngroups = (nw + NRING - 1) // NRING
        cur_row, cnt = lax.fori_loop(0, ngroups, group_body,
                                     (pass_base, jnp.int32(0)))

        @pl.when(cnt > 0)
        def _():
            finalize(cur_row, cnt, pass_base)

        pltpu.sync_copy(si_v, oi_hbm.at[pl.ds(pass_base * 64, P * 64)])
        pltpu.sync_copy(sw_v, ow_hbm.at[pl.ds(pass_base * 64, P * 64)])
        return 0

    lax.fori_loop(0, NPASS, pass_body, 0)


def _build_sc_select():
    return functools.partial(
        pl.kernel,
        out_type=[
            jax.ShapeDtypeStruct((M_VOXELS * MAX_NEIGHBORS,), jnp.int32),
            jax.ShapeDtypeStruct((M_VOXELS * MAX_NEIGHBORS,), jnp.float32),
        ],
        mesh=plsc.VectorSubcoreMesh(core_axis_name="c", subcore_axis_name="s",
                                    num_cores=2, num_subcores=16),
        compiler_params=pltpu.CompilerParams(needs_layout_passes=False),
        scratch_types=[
            pltpu.VMEM((P * NBLK,), jnp.float32),
            pltpu.VMEM((P * NBLK + 2 * 16,), jnp.int32),
            pltpu.VMEM((WAVE, 128), jnp.float32),
            pltpu.VMEM((WAVE, 128), jnp.float32),
            pltpu.VMEM((WAVE, 128), jnp.float32),
            pltpu.VMEM((WAVE, 128), jnp.float32),
            pltpu.VMEM((CANDCAP,), jnp.float32),
            pltpu.VMEM((CANDCAP,), jnp.int32),
            pltpu.VMEM((64,), jnp.float32),
            pltpu.VMEM((64,), jnp.int32),
            pltpu.VMEM((P * MAX_NEIGHBORS,), jnp.int32),
            pltpu.VMEM((P * MAX_NEIGHBORS,), jnp.float32),
            pltpu.SemaphoreType.DMA,
            pltpu.SemaphoreType.DMA,
            pltpu.SemaphoreType.DMA,
            pltpu.SemaphoreType.DMA,
        ],
    )(_sc_select_body)


def kernel(mu, scale, rotation, features, voxel_coords, point_cloud_range):
    vc = voxel_coords.astype(jnp.int32)
    mu_t = mu.T
    sc_t = scale.T
    pcr = point_cloud_range.reshape(1, 6)

    scores, bm, centers = pl.pallas_call(
        _score_kernel,
        grid=(M_VOXELS // ROWS, N_GAUSS // GB),
        in_specs=[
            pl.BlockSpec((ROWS, 4), lambda i, j: (i, 0)),
            pl.BlockSpec((3, GB), lambda i, j: (0, j)),
            pl.BlockSpec((3, GB), lambda i, j: (0, j)),
            pl.BlockSpec((1, 6), lambda i, j: (0, 0)),
        ],
        out_specs=[
            pl.BlockSpec((ROWS, GB), lambda i, j: (i, j)),
            pl.BlockSpec((1, ROWS, GB // 128), lambda i, j: (j, i, 0)),
            pl.BlockSpec((ROWS, 3), lambda i, j: (i, 0)),
        ],
        out_shape=[
            jax.ShapeDtypeStruct((M_VOXELS, N_GAUSS), jnp.float32),
            jax.ShapeDtypeStruct((N_GAUSS // GB, M_VOXELS, GB // 128),
                                 jnp.float32),
            jax.ShapeDtypeStruct((M_VOXELS, 3), jnp.float32),
        ],
    )(vc, mu_t, sc_t, pcr)

    bm_flat = jnp.transpose(bm, (1, 0, 2)).reshape(M_VOXELS * NBLK)
    oi, ow = _build_sc_select()(scores.reshape(M_VOXELS * NBLK, 128),
                                bm_flat)
    idx = oi.reshape(M_VOXELS, MAX_NEIGHBORS)
    w = ow.reshape(M_VOXELS, MAX_NEIGHBORS)
    masks = idx >= 0
    return centers, idx, w, masks


# speedup vs baseline: 61.9677x; 1.9902x over previous
import functools

import jax
import jax.numpy as jnp
from jax import lax
from jax.experimental import pallas as pl
from jax.experimental.pallas import tpu as pltpu
from jax.experimental.pallas import tpu_sc as plsc

VOXEL_SIZE = (0.05, 0.05, 0.05)
SCALE_MULTIPLIER = 3.0
MAX_NEIGHBORS = 64
N_GAUSS = 8192
M_VOXELS = 16384

ROWS = 256
GB = 1024
NBLK = 64
INF = float("inf")

NWORK = 32
RW = M_VOXELS // NWORK
P = 256
NPASS = RW // P
WAVE = 32
NRING = 4
CANDCAP = N_GAUSS + 64


def _score_kernel(vc_ref, mu_ref, sc_ref, pcr_ref, scores_ref, bm_ref, cent_ref):
    x = vc_ref[:, 3:4].astype(jnp.float32)
    y = vc_ref[:, 2:3].astype(jnp.float32)
    z = vc_ref[:, 1:2].astype(jnp.float32)
    cx = pcr_ref[0, 0] + (x + 0.5) * VOXEL_SIZE[0]
    cy = pcr_ref[0, 1] + (y + 0.5) * VOXEL_SIZE[1]
    cz = pcr_ref[0, 2] + (z + 0.5) * VOXEL_SIZE[2]
    cent_ref[...] = jnp.concatenate([cx, cy, cz], axis=1)

    zeros_slab = jnp.zeros((1, 8, 128), jnp.float32)
    cxb = cx[:, :, None] + zeros_slab
    cyb = cy[:, :, None] + zeros_slab
    czb = cz[:, :, None] + zeros_slab

    bms = []
    for jj in range(NBLK // 8):
        sl = slice(jj * 8, (jj + 1) * 8)
        mux = mu_ref[0:1, sl, :]
        muy = mu_ref[1:2, sl, :]
        muz = mu_ref[2:3, sl, :]
        sx = sc_ref[0:1, sl, :]
        sy = sc_ref[1:2, sl, :]
        sz = sc_ref[2:3, sl, :]
        s2x = sx * sx
        s2y = sy * sy
        s2z = sz * sz
        ivx = 1.0 / (s2x + 1e-8)
        ivy = 1.0 / (s2y + 1e-8)
        ivz = 1.0 / (s2z + 1e-8)
        r = SCALE_MULTIPLIER * jnp.sqrt(s2x + s2y + s2z)
        r2 = r * r

        dx = cxb - mux
        dy = cyb - muy
        dz = czb - muz
        d2 = dx * dx * ivx + dy * dy * ivy + dz * dz * ivz
        md2 = jnp.where(d2 <= r2, d2, INF)
        scores_ref[:, sl, :] = md2
        bms.append(jnp.min(md2, axis=2))
    bm_ref[...] = jnp.concatenate(bms, axis=1)


def _splat_i32(x):
    return jnp.full((16,), x, jnp.int32)


def _splat_f32(x):
    return jnp.full((16,), x, jnp.float32)


def _mask_i32(mask):
    return jnp.where(mask, _splat_i32(1), _splat_i32(0))


def _vsort(k, v):
    return plsc.sort_key_val(k, v)


def _rev2(k, v):
    return lax.rev(k, (0,)), lax.rev(v, (0,))


def _cmpsel(ak, av, bk, bv):
    m = ak <= bk
    lk = jnp.where(m, ak, bk)
    lv = jnp.where(m, av, bv)
    hk = jnp.where(m, bk, ak)
    hv = jnp.where(m, bv, av)
    return lk, lv, hk, hv


def _merge16(ak, av, bk, bv):
    rbk, rbv = _rev2(bk, bv)
    lk, lv, hk, hv = _cmpsel(ak, av, rbk, rbv)
    lk, lv = _vsort(lk, lv)
    hk, hv = _vsort(hk, hv)
    return lk, lv, hk, hv


def _bitonic2(x0k, x0v, x1k, x1v):
    lk, lv, hk, hv = _cmpsel(x0k, x0v, x1k, x1v)
    lk, lv = _vsort(lk, lv)
    hk, hv = _vsort(hk, hv)
    return lk, lv, hk, hv


def _sort64(k, v):
    s = [_vsort(k[i], v[i]) for i in range(4)]
    a0k, a0v, a1k, a1v = _merge16(*s[0], *s[1])
    b0k, b0v, b1k, b1v = _merge16(*s[2], *s[3])
    rb0k, rb0v = _rev2(b1k, b1v)
    rb1k, rb1v = _rev2(b0k, b0v)
    l0k, l0v, h0k, h0v = _cmpsel(a0k, a0v, rb0k, rb0v)
    l1k, l1v, h1k, h1v = _cmpsel(a1k, a1v, rb1k, rb1v)
    s0k, s0v, s1k, s1v = _bitonic2(l0k, l0v, l1k, l1v)
    s2k, s2v, s3k, s3v = _bitonic2(h0k, h0v, h1k, h1v)
    return (s0k, s1k, s2k, s3k), (s0v, s1v, s2v, s3v)


def _insert16(sk, sv, tk, tv):
    rtk, rtv = _rev2(tk, tv)
    mk, mv, _, _ = _cmpsel(sk[3], sv[3], rtk, rtv)
    t2k, t2v = _vsort(mk, mv)
    r2k, r2v = _rev2(t2k, t2v)
    lk, lv, hk, hv = _cmpsel(sk[2], sv[2], r2k, r2v)
    n3k, n3v = _vsort(hk, hv)
    t3k, t3v = _vsort(lk, lv)
    r3k, r3v = _rev2(t3k, t3v)
    lk, lv, hk, hv = _cmpsel(sk[1], sv[1], r3k, r3v)
    n2k, n2v = _vsort(hk, hv)
    t4k, t4v = _vsort(lk, lv)
    r4k, r4v = _rev2(t4k, t4v)
    lk, lv, hk, hv = _cmpsel(sk[0], sv[0], r4k, r4v)
    n1k, n1v = _vsort(hk, hv)
    n0k, n0v = _vsort(lk, lv)
    return (n0k, n1k, n2k, n3k), (n0v, n1v, n2v, n3v)


def _popcount(mask):
    return plsc.all_reduce_population_count(mask)[0]


def _sc_select_body(scores_hbm, bm_hbm, oi_hbm, ow_hbm,
                    bm_v, list_v, rb0, rb1, rb2, rb3,
                    ck_v, ci_v, kk_v, vv_v, si_v, sw_v,
                    sem0, sem1, sem2, sem3):
    rings = (rb0, rb1, rb2, rb3)
    sems = (sem0, sem1, sem2, sem3)
    wid = lax.axis_index("s") * 2 + lax.axis_index("c")

    def finalize(cur_row, cnt, pass_base):
        r_local = cur_row - pass_base
        for q in range(4):
            ck_v[pl.ds(cnt + q * 16, 16)] = jnp.full((16,), INF, jnp.float32)
        k4 = [ck_v[pl.ds(i * 16, 16)] for i in range(4)]
        v4 = [ci_v[pl.ds(i * 16, 16)] for i in range(4)]
        sk, sv = _sort64(k4, v4)
        nch = jnp.maximum(0, (cnt - 49) // 16)
        for j in range(4):
            kk_v[pl.ds(j * 16, 16)] = sk[j]
            vv_v[pl.ds(j * 16, 16)] = sv[j]

        def chunk_body(m, c):
            off = 64 + m * 16
            tk, tv = _vsort(ck_v[pl.ds(off, 16)], ci_v[pl.ds(off, 16)])
            csk = [kk_v[pl.ds(j * 16, 16)] for j in range(4)]
            csv = [vv_v[pl.ds(j * 16, 16)] for j in range(4)]
            nk, nv = _insert16(csk, csv, tk, tv)
            for j in range(4):
                kk_v[pl.ds(j * 16, 16)] = nk[j]
                vv_v[pl.ds(j * 16, 16)] = nv[j]
            return c
        lax.fori_loop(0, nch, chunk_body, 0)

        for j in range(4):
            fk = kk_v[pl.ds(j * 16, 16)]
            fv = vv_v[pl.ds(j * 16, 16)]
            valid = fk < INF
            si_v[pl.ds(r_local * 64 + j * 16, 16)] = jnp.where(
                valid, fv, jnp.full((16,), -1, jnp.int32))
            sw_v[pl.ds(r_local * 64 + j * 16, 16)] = jnp.where(
                valid, fk, jnp.full((16,), 0.0, jnp.float32))

    def pass_body(pi, _):
        pass_base = wid * RW + pi * P

        def init_body(t, c):
            si_v[pl.ds(t * 16, 16)] = jnp.full((16,), -1, jnp.int32)
            sw_v[pl.ds(t * 16, 16)] = jnp.zeros((16,), jnp.float32)
            return c
        lax.fori_loop(0, P * 64 // 16, init_body, 0)

        pltpu.sync_copy(bm_hbm.at[pl.ds(pass_base * 64, P * 64)], bm_v)

        def brow(r, off):
            for j in range(4):
                bmv = bm_v[pl.ds(r * 64 + j * 16, 16)]
                mask = bmv < INF
                pc = _popcount(mask)
                ids = ((pass_base + r) * 64 + j * 16
                       + lax.broadcasted_iota(jnp.int32, (16,), 0))
                plsc.store_compressed(list_v.at[pl.ds(off, 16)], ids,
                                      mask=mask)
                off = off + pc
            return off

        num_list = lax.fori_loop(0, P, brow, jnp.int32(0))
        list_v[pl.ds(num_list, 16)] = jnp.zeros((16,), jnp.int32)
        list_v[pl.ds(num_list + 16, 16)] = jnp.zeros((16,), jnp.int32)
        nw = (num_list + WAVE - 1) // WAVE

        for s in range(NRING):
            @pl.when(s < nw)
            def _():
                pltpu.async_copy(
                    scores_hbm.at[list_v.at[pl.ds(s * WAVE, WAVE)]],
                    rings[s], sems[s])

        def group_body(g, carry):
            for s in range(NRING):
                w = g * NRING + s

                def do_wave(c, s=s, w=w):
                    pltpu.make_async_copy(
                        scores_hbm.at[list_v.at[pl.ds(0, WAVE)]],
                        rings[s], sems[s]).wait()
                    nblk = jnp.minimum(WAVE, num_list - w * WAVE)

                    def blk_body(b, cc):
                        cur_row, cnt = cc
                        bid = list_v[pl.ds(w * WAVE + b, 16)][0]
                        row = bid // 64
                        blk = bid % 64

                        def rowchange(cr, cn):
                            @pl.when(cn > 0)
                            def _():
                                finalize(cr, cn, pass_base)
                            return row, jnp.int32(0)

                        cur_row, cnt = lax.cond(row != cur_row, rowchange,
                                                lambda cr, cn: (cr, cn),
                                                cur_row, cnt)
                        rowref = rings[s].at[b]
                        xvs = []
                        msks = []
                        offs = [cnt]
                        for v8 in range(8):
                            xv = rowref[pl.ds(v8 * 16, 16)]
                            mask = xv < INF
                            xvs.append(xv)
                            msks.append(mask)
                            offs.append(offs[-1] + _popcount(mask))
                        for v8 in range(8):
                            gidx = (blk * 128 + v8 * 16
                                    + lax.broadcasted_iota(jnp.int32, (16,), 0))
                            plsc.store_compressed(ck_v.at[pl.ds(offs[v8], 16)],
                                                  xvs[v8], mask=msks[v8])
                            plsc.store_compressed(ci_v.at[pl.ds(offs[v8], 16)],
                                                  gidx, mask=msks[v8])
                        return cur_row, offs[8]

                    c = lax.fori_loop(0, nblk, blk_body, c)

                    @pl.when(w + NRING < nw)
                    def _():
                        pltpu.async_copy(
                            scores_hbm.at[
                                list_v.at[pl.ds((w + NRING) * WAVE, WAVE)]],
                            rings[s], sems[s])
                    return c

                carry = lax.cond(w < nw, do_wave, lambda c: c, carry)
            return carry

        ngroups = (nw + NRING - 1) // NRING
        cur_row, cnt = lax.fori_loop(0, ngroups, group_body,
                                     (pass_base, jnp.int32(0)))

        @pl.when(cnt > 0)
        def _():
            finalize(cur_row, cnt, pass_base)

        pltpu.sync_copy(si_v, oi_hbm.at[pl.ds(pass_base * 64, P * 64)])
        pltpu.sync_copy(sw_v, ow_hbm.at[pl.ds(pass_base * 64, P * 64)])
        return 0

    lax.fori_loop(0, NPASS, pass_body, 0)


def _build_sc_select():
    return functools.partial(
        pl.kernel,
        out_type=[
            jax.ShapeDtypeStruct((M_VOXELS * MAX_NEIGHBORS,), jnp.int32),
            jax.ShapeDtypeStruct((M_VOXELS * MAX_NEIGHBORS,), jnp.float32),
        ],
        mesh=plsc.VectorSubcoreMesh(core_axis_name="c", subcore_axis_name="s",
                                    num_cores=2, num_subcores=16),
        compiler_params=pltpu.CompilerParams(needs_layout_passes=False),
        scratch_types=[
            pltpu.VMEM((P * NBLK,), jnp.float32),
            pltpu.VMEM((P * NBLK + 2 * 16,), jnp.int32),
            pltpu.VMEM((WAVE, 128), jnp.float32),
            pltpu.VMEM((WAVE, 128), jnp.float32),
            pltpu.VMEM((WAVE, 128), jnp.float32),
            pltpu.VMEM((WAVE, 128), jnp.float32),
            pltpu.VMEM((CANDCAP,), jnp.float32),
            pltpu.VMEM((CANDCAP,), jnp.int32),
            pltpu.VMEM((64,), jnp.float32),
            pltpu.VMEM((64,), jnp.int32),
            pltpu.VMEM((P * MAX_NEIGHBORS,), jnp.int32),
            pltpu.VMEM((P * MAX_NEIGHBORS,), jnp.float32),
            pltpu.SemaphoreType.DMA,
            pltpu.SemaphoreType.DMA,
            pltpu.SemaphoreType.DMA,
            pltpu.SemaphoreType.DMA,
        ],
    )(_sc_select_body)


def kernel(mu, scale, rotation, features, voxel_coords, point_cloud_range):
    vc = voxel_coords.astype(jnp.int32)
    mu_t = mu.T.reshape(3, NBLK, 128)
    sc_t = scale.T.reshape(3, NBLK, 128)
    pcr = point_cloud_range.reshape(1, 6)

    scores, bm, centers = pl.pallas_call(
        _score_kernel,
        grid=(M_VOXELS // ROWS,),
        in_specs=[
            pl.BlockSpec((ROWS, 4), lambda i: (i, 0)),
            pl.BlockSpec((3, NBLK, 128), lambda i: (0, 0, 0)),
            pl.BlockSpec((3, NBLK, 128), lambda i: (0, 0, 0)),
            pl.BlockSpec((1, 6), lambda i: (0, 0)),
        ],
        out_specs=[
            pl.BlockSpec((ROWS, NBLK, 128), lambda i: (i, 0, 0)),
            pl.BlockSpec((ROWS, NBLK), lambda i: (i, 0)),
            pl.BlockSpec((ROWS, 3), lambda i: (i, 0)),
        ],
        out_shape=[
            jax.ShapeDtypeStruct((M_VOXELS, NBLK, 128), jnp.float32),
            jax.ShapeDtypeStruct((M_VOXELS, NBLK), jnp.float32),
            jax.ShapeDtypeStruct((M_VOXELS, 3), jnp.float32),
        ],
    )(vc, mu_t, sc_t, pcr)

    bm_flat = bm.reshape(M_VOXELS * NBLK)
    oi, ow = _build_sc_select()(scores.reshape(M_VOXELS * NBLK, 128),
                                bm_flat)
    idx = oi.reshape(M_VOXELS, MAX_NEIGHBORS)
    w = ow.reshape(M_VOXELS, MAX_NEIGHBORS)
    masks = idx >= 0
    return centers, idx, w, masks
